# baseline (speedup 1.0000x reference)
import functools

import jax
import jax.numpy as jnp
from jax import lax
from jax.experimental import pallas as pl
from jax.experimental.pallas import tpu as pltpu
from jax.experimental.pallas import tpu_sc as plsc

_NC, _NS, _LANES = 2, 16, 16
_NW = _NC * _NS

_NG = 3
_CH = 8
_UNROLL = 4


def _sc_add(nbatch, nseq, d):
    seq_per_w = nseq // _NW
    nchunk = seq_per_w // _CH
    assert d & (d - 1) == 0
    dshift = d.bit_length() - 1

    mesh = plsc.VectorSubcoreMesh(core_axis_name="c", subcore_axis_name="s")

    @functools.partial(
        pl.kernel,
        mesh=mesh,
        out_type=jax.ShapeDtypeStruct((nbatch * nseq, d), jnp.float32),
        scratch_types=[
            pltpu.VMEM((_NG, nbatch, _CH, d), jnp.float32),
            pltpu.VMEM((_NG, _CH, d), jnp.float32),
        ] + [pltpu.SemaphoreType.DMA] * (3 * _NG),
    )
    def body(x_hbm, p_hbm, o_hbm, xbuf, pbuf, *sems):
        lsems = sems[:_NG]
        psems = sems[_NG:2 * _NG]
        ssems = sems[2 * _NG:]
        c = lax.axis_index("c")
        s = lax.axis_index("s")
        wid = s * _NC + c
        base = wid * seq_per_w

        def issue_group(k):
            g = k % _NG
            hs = [pltpu.async_copy(p_hbm.at[pl.ds(base + k * _CH, _CH)],
                                   pbuf.at[g], psems[g])]
            for b in range(nbatch):
                roff = b * nseq + base + k * _CH
                hs.append(pltpu.async_copy(x_hbm.at[pl.ds(roff, _CH)],
                                           xbuf.at[g, b], lsems[g]))
            return hs

        loads = {k: issue_group(k) for k in range(min(_NG - 1, nchunk))}
        stores = {}
        for k in range(nchunk):
            g = k % _NG
            for h in loads.pop(k):
                h.wait()

            @plsc.parallel_loop(0, _CH * d, step=_LANES, unroll=_UNROLL)
            def cbody(o, g=g):
                r = o >> dshift
                sl = pl.ds(pl.multiple_of(o & (d - 1), _LANES), _LANES)
                p = pbuf[g, r, sl]
                for b in range(nbatch):
                    xbuf[g, b, r, sl] = xbuf[g, b, r, sl] + p

            shs = []
            for b in range(nbatch):
                roff = b * nseq + base + k * _CH
                shs.append(pltpu.async_copy(xbuf.at[g, b],
                                            o_hbm.at[pl.ds(roff, _CH)],
                                            ssems[g]))
            stores[k] = shs

            kn = k + _NG - 1
            if kn < nchunk:
                if k - 1 >= 0:
                    for h in stores.pop(k - 1):
                        h.wait()
                loads[kn] = issue_group(kn)
        for k in sorted(stores):
            for h in stores[k]:
                h.wait()

    return body


def kernel(x, pos_table):
    B, L, D = x.shape
    xf = x.reshape(B * L, D)
    out = _sc_add(B, L, D)(xf, pos_table)
    return out.reshape(B, L, D)
